# baseline (speedup 1.0000x reference)
import functools

import jax
import jax.numpy as jnp
from jax.experimental import pallas as pl


def _rvq_body(n_cb, K, x_ref, wt_ref, b2_ref, codes_ref, q_ref, loss_ref):
    i_blk = pl.program_id(0)

    @pl.when(i_blk == 0)
    def _():
        loss_ref[...] = jnp.zeros((1, 1), jnp.float32)

    x = x_ref[...]
    r = x
    qsum = jnp.zeros_like(x)
    loss = jnp.float32(0.0)
    PB = x.shape[0]
    for i in range(n_cb):
        wt_i = wt_ref[i]
        ab = jnp.dot(r, wt_i, preferred_element_type=jnp.float32)
        a2 = jnp.sum(r * r, axis=1, keepdims=True)
        d2 = a2 + b2_ref[i][None, :] - 2.0 * ab
        dist = jnp.sqrt(jnp.maximum(d2, 0.0))
        idx = jnp.argmin(dist, axis=1)
        codes_ref[0, i, :] = idx
        oh = (jax.lax.broadcasted_iota(jnp.int32, (PB, K), 1)
              == idx[:, None]).astype(jnp.float32)
        q = jax.lax.dot_general(
            oh, wt_i, (((1,), (1,)), ((), ())),
            precision=jax.lax.Precision.HIGHEST,
            preferred_element_type=jnp.float32)
        qsum = qsum + q
        r = r - q
        loss = loss + jnp.sum(r * r)
    q_ref[...] = x + (qsum - x)
    loss_ref[...] += jnp.reshape(loss, (1, 1))


def kernel(embeddings, codebooks):
    B, T, D = embeddings.shape
    n_cb, K, _ = codebooks.shape
    N = B * T
    PB = 480

    x = embeddings.reshape(N, D)
    wt = jnp.swapaxes(codebooks, 1, 2)
    b2 = jnp.stack([jnp.sum(codebooks[i] * codebooks[i], axis=1)
                    for i in range(n_cb)])

    grid = (N // PB,)
    codes_t, q, loss = pl.pallas_call(
        functools.partial(_rvq_body, n_cb, K),
        grid=grid,
        in_specs=[
            pl.BlockSpec((PB, D), lambda i: (i, 0)),
            pl.BlockSpec((n_cb, D, K), lambda i: (0, 0, 0)),
            pl.BlockSpec((n_cb, K), lambda i: (0, 0)),
        ],
        out_specs=[
            pl.BlockSpec((1, n_cb, PB), lambda i: (i, 0, 0)),
            pl.BlockSpec((PB, D), lambda i: (i, 0)),
            pl.BlockSpec((1, 1), lambda i: (0, 0)),
        ],
        out_shape=[
            jax.ShapeDtypeStruct((N // PB, n_cb, PB), jnp.int32),
            jax.ShapeDtypeStruct((N, D), jnp.float32),
            jax.ShapeDtypeStruct((1, 1), jnp.float32),
        ],
    )(x, wt, b2)

    codes = jnp.swapaxes(codes_t, 1, 2).reshape(B, T, n_cb)
    quantized = q.reshape(B, T, D)
    total_loss = loss[0, 0] / jnp.float32(N * D) / jnp.float32(n_cb)
    return codes, quantized, total_loss


# speedup vs baseline: 1.4435x; 1.4435x over previous
import functools

import jax
import jax.numpy as jnp
from jax.experimental import pallas as pl

_NT = (((1,), (1,)), ((), ()))
_NN = (((1,), (0,)), ((), ()))


def _level_body(K, r_ref, a2_ref, whi_ref, wmid_ref, wlo_ref, b2_ref,
                codes_ref, q_ref, rn_ref, loss_ref):
    i_blk = pl.program_id(0)

    @pl.when(i_blk == 0)
    def _():
        loss_ref[...] = jnp.zeros((1, 1), jnp.float32)

    f32 = jnp.float32
    r = r_ref[...]
    PB = r.shape[0]
    rb = r.astype(jnp.bfloat16)
    w_i = whi_ref[...]
    ab = jax.lax.dot_general(rb, w_i, _NT, preferred_element_type=f32)
    d2 = a2_ref[...] + b2_ref[...] - 2.0 * ab
    dist = jnp.sqrt(jnp.maximum(d2, 0.0))
    idx = jnp.argmin(dist, axis=1)
    codes_ref[0, 0, :] = idx
    oh = (jax.lax.broadcasted_iota(jnp.int32, (PB, K), 1)
          == idx[:, None]).astype(jnp.bfloat16)
    q = ((jax.lax.dot_general(oh, w_i, _NN, preferred_element_type=f32)
          + jax.lax.dot_general(oh, wmid_ref[...], _NN,
                                preferred_element_type=f32))
         + jax.lax.dot_general(oh, wlo_ref[...], _NN,
                               preferred_element_type=f32))
    q_ref[...] = q
    rn = r - q
    rn_ref[...] = rn
    loss_ref[...] += jnp.reshape(jnp.sum(rn * rn), (1, 1))


def _level_call(K, N, D, PB, r, a2, whi_t, wmid_t, wlo_t, b2):
    return pl.pallas_call(
        functools.partial(_level_body, K),
        grid=(N // PB,),
        in_specs=[
            pl.BlockSpec((PB, D), lambda i: (i, 0)),
            pl.BlockSpec((PB, 1), lambda i: (i, 0)),
            pl.BlockSpec((K, D), lambda i: (0, 0)),
            pl.BlockSpec((K, D), lambda i: (0, 0)),
            pl.BlockSpec((K, D), lambda i: (0, 0)),
            pl.BlockSpec((1, K), lambda i: (0, 0)),
        ],
        out_specs=[
            pl.BlockSpec((1, 1, PB), lambda i: (i, 0, 0)),
            pl.BlockSpec((PB, D), lambda i: (i, 0)),
            pl.BlockSpec((PB, D), lambda i: (i, 0)),
            pl.BlockSpec((1, 1), lambda i: (0, 0)),
        ],
        out_shape=[
            jax.ShapeDtypeStruct((N // PB, 1, PB), jnp.int32),
            jax.ShapeDtypeStruct((N, D), jnp.float32),
            jax.ShapeDtypeStruct((N, D), jnp.float32),
            jax.ShapeDtypeStruct((1, 1), jnp.float32),
        ],
    )(r, a2, whi_t, wmid_t, wlo_t, b2)


def kernel(embeddings, codebooks):
    B, T, D = embeddings.shape
    n_cb, K, _ = codebooks.shape
    N = B * T
    PB = 480

    x = embeddings.reshape(N, D)
    whi = codebooks.astype(jnp.bfloat16)
    e1 = codebooks - whi.astype(jnp.float32)
    wmid = e1.astype(jnp.bfloat16)
    wlo = (e1 - wmid.astype(jnp.float32)).astype(jnp.bfloat16)

    r = x
    qsum = None
    loss = jnp.float32(0.0)
    codes_list = []
    for i in range(n_cb):
        w = codebooks[i]
        b2 = jnp.sum(w * w, axis=1)
        a2 = jnp.sum(r * r, axis=1, keepdims=True)
        codes_i, q, r, partial = _level_call(
            K, N, D, PB, r, a2, whi[i], wmid[i], wlo[i], b2.reshape(1, K))
        codes_list.append(codes_i.reshape(N))
        qsum = q if qsum is None else qsum + q
        loss = loss + partial[0, 0]
    codes = jnp.stack([c.reshape(B, T) for c in codes_list], axis=-1)
    quantized = (x + (qsum - x)).reshape(B, T, D)
    total_loss = loss / jnp.float32(N * D) / jnp.float32(n_cb)
    return codes, quantized, total_loss
